# baseline (speedup 1.0000x reference)
import functools

import jax
import jax.numpy as jnp
from jax import lax
from jax.experimental import pallas as pl
from jax.experimental.pallas import tpu as pltpu
from jax.experimental.pallas import tpu_sc as plsc

N_NODES = 10000
N_EDGES = 160000
IN_DIM = 500
HID = 8

NC = 2
NS = 16
NW = NC * NS

CHUNK = 5000
NCHUNK = 1
EPW = CHUNK * NCHUNK
NPAD = 10112
SLAB = NPAD // NS
NROW128 = NPAD * HID // 128


def _tc_linear_body(xt_ref, w_ref, b_ref, o_ref):
    o_ref[: xt_ref.shape[1], :] = (
        lax.dot_general(
            xt_ref[...], w_ref[...],
            dimension_numbers=(((0,), (0,)), ((), ())),
            preferred_element_type=jnp.float32,
        )
        + b_ref[...]
    )


def _tc_linear(xt, w, b):
    return pl.pallas_call(
        _tc_linear_body,
        out_shape=jax.ShapeDtypeStruct((NPAD, HID), jnp.float32),
    )(xt, w, b)


def _tc_combine_linear_body(p_ref, w_ref, b_ref, o_ref):
    h = jnp.maximum(p_ref[0] + p_ref[1], 0.0)
    o_ref[...] = (
        jnp.dot(h, w_ref[...], preferred_element_type=jnp.float32) + b_ref[...]
    )


def _tc_combine_linear(p, w, b):
    return pl.pallas_call(
        _tc_combine_linear_body,
        out_shape=jax.ShapeDtypeStruct((NROW128, 128), jnp.float32),
    )(p, w, b)


def _tc_combine_body(p_ref, o_ref):
    o_ref[...] = p_ref[0] + p_ref[1]


def _tc_combine(p):
    return pl.pallas_call(
        _tc_combine_body,
        out_shape=jax.ShapeDtypeStruct((NROW128, 128), jnp.float32),
    )(p)


def _sc_agg_body(table_hbm, edges_hbm, zeros_hbm, out_hbm,
                 acc_sp, src_v, dst_v, chunk_v, sem_g, sem_s):
    c = lax.axis_index("c")
    s = lax.axis_index("s")
    wid = c * NS + s
    slab = pl.ds(s * SLAB, SLAB)

    pltpu.sync_copy(edges_hbm.at[0].at[wid], src_v)
    pltpu.sync_copy(edges_hbm.at[1].at[wid], dst_v)
    for j in range(NCHUNK):
        pltpu.async_copy(table_hbm.at[src_v.at[j]], chunk_v.at[j], sem_g)
    pltpu.sync_copy(zeros_hbm.at[slab], acc_sp.at[slab])
    plsc.subcore_barrier()

    for j in range(NCHUNK):
        pltpu.make_async_copy(table_hbm.at[src_v.at[j]], chunk_v.at[j],
                              sem_g).wait()
        pltpu.async_copy(chunk_v.at[j], acc_sp.at[dst_v.at[j]], sem_s,
                         add=True)
    for j in range(NCHUNK):
        pltpu.make_async_copy(chunk_v.at[j], acc_sp.at[dst_v.at[j]],
                              sem_s).wait()
    plsc.subcore_barrier()

    pltpu.sync_copy(acc_sp.at[slab], out_hbm.at[c].at[slab])


_sc_agg = functools.partial(
    pl.kernel,
    out_type=jax.ShapeDtypeStruct((NC, NPAD, HID), jnp.float32),
    mesh=plsc.VectorSubcoreMesh(
        core_axis_name="c", subcore_axis_name="s", num_cores=NC,
        num_subcores=NS,
    ),
    compiler_params=pltpu.CompilerParams(use_tc_tiling_on_sc=False),
    scratch_types=[
        pltpu.VMEM_SHARED((NPAD, HID), jnp.float32),
        pltpu.VMEM((NCHUNK, CHUNK), jnp.int32),
        pltpu.VMEM((NCHUNK, CHUNK), jnp.int32),
        pltpu.VMEM((NCHUNK, CHUNK, HID), jnp.float32),
        pltpu.SemaphoreType.DMA,
        pltpu.SemaphoreType.DMA,
    ],
)(_sc_agg_body)


def kernel(x, edge_index, W1, b1, W2, b2):
    edges = edge_index.astype(jnp.int32).reshape(2, NW, NCHUNK, CHUNK)

    zeros = jnp.zeros((NPAD, HID), jnp.float32)
    w2p = jnp.pad(W2, ((0, 0), (0, HID - W2.shape[1])))
    w2bd = jnp.kron(jnp.eye(16, dtype=jnp.float32), w2p)
    b1r = b1.reshape(1, HID)
    b2r = jnp.tile(jnp.pad(b2, (0, HID - b2.shape[0])), 16).reshape(1, 128)

    support1 = _tc_linear(x.T, W1, b1r)
    p1 = _sc_agg(support1, edges, zeros)

    p1v = p1.reshape(NC, NROW128, 128)
    support2 = _tc_combine_linear(p1v, w2bd, b2r).reshape(NPAD, HID)
    p2 = _sc_agg(support2, edges, zeros)

    out = _tc_combine(p2.reshape(NC, NROW128, 128)).reshape(NPAD, HID)
    return out[:N_NODES, : W2.shape[1]]


# speedup vs baseline: 1.0449x; 1.0449x over previous
import functools

import jax
import jax.numpy as jnp
from jax import lax
from jax.experimental import pallas as pl
from jax.experimental.pallas import tpu as pltpu
from jax.experimental.pallas import tpu_sc as plsc

N_NODES = 10000
N_EDGES = 160000
IN_DIM = 500
HID = 8

NC = 2
NS = 16
NW = NC * NS

CHUNK = 1000
NCHUNK = 5
EPW = CHUNK * NCHUNK
NPAD = 10112
SLAB = NPAD // NS
NROW128 = NPAD * HID // 128


def _tc_linear_body(xt_ref, w_ref, b_ref, o_ref):
    o_ref[: xt_ref.shape[1], :] = (
        lax.dot_general(
            xt_ref[...], w_ref[...],
            dimension_numbers=(((0,), (0,)), ((), ())),
            preferred_element_type=jnp.float32,
        )
        + b_ref[...]
    )


def _tc_linear(xt, w, b):
    return pl.pallas_call(
        _tc_linear_body,
        out_shape=jax.ShapeDtypeStruct((NPAD, HID), jnp.float32),
    )(xt, w, b)


def _tc_combine_linear_body(p_ref, w_ref, b_ref, o_ref):
    h = jnp.maximum(p_ref[0] + p_ref[1], 0.0)
    o_ref[...] = (
        jnp.dot(h, w_ref[...], preferred_element_type=jnp.float32) + b_ref[...]
    )


def _tc_combine_linear(p, w, b):
    return pl.pallas_call(
        _tc_combine_linear_body,
        out_shape=jax.ShapeDtypeStruct((NROW128, 128), jnp.float32),
    )(p, w, b)


def _tc_combine_body(p_ref, o_ref):
    o_ref[...] = p_ref[0] + p_ref[1]


def _tc_combine(p):
    return pl.pallas_call(
        _tc_combine_body,
        out_shape=jax.ShapeDtypeStruct((NROW128, 128), jnp.float32),
    )(p)


def _sc_agg_body(table_hbm, edges_hbm, zeros_hbm, out_hbm,
                 acc_sp, src_v, dst_v, chunk_v, sem_g, sem_s):
    c = lax.axis_index("c")
    s = lax.axis_index("s")
    wid = c * NS + s
    slab = pl.ds(s * SLAB, SLAB)

    pltpu.sync_copy(edges_hbm.at[0].at[wid], src_v)
    pltpu.sync_copy(edges_hbm.at[1].at[wid], dst_v)
    for j in range(NCHUNK):
        pltpu.async_copy(table_hbm.at[src_v.at[j]], chunk_v.at[j], sem_g)
    pltpu.sync_copy(zeros_hbm.at[slab], acc_sp.at[slab])
    plsc.subcore_barrier()

    for j in range(NCHUNK):
        pltpu.make_async_copy(table_hbm.at[src_v.at[j]], chunk_v.at[j],
                              sem_g).wait()
        pltpu.async_copy(chunk_v.at[j], acc_sp.at[dst_v.at[j]], sem_s,
                         add=True)
    for j in range(NCHUNK):
        pltpu.make_async_copy(chunk_v.at[j], acc_sp.at[dst_v.at[j]],
                              sem_s).wait()
    plsc.subcore_barrier()

    pltpu.sync_copy(acc_sp.at[slab], out_hbm.at[c].at[slab])


_sc_agg = functools.partial(
    pl.kernel,
    out_type=jax.ShapeDtypeStruct((NC, NPAD, HID), jnp.float32),
    mesh=plsc.VectorSubcoreMesh(
        core_axis_name="c", subcore_axis_name="s", num_cores=NC,
        num_subcores=NS,
    ),
    compiler_params=pltpu.CompilerParams(use_tc_tiling_on_sc=False),
    scratch_types=[
        pltpu.VMEM_SHARED((NPAD, HID), jnp.float32),
        pltpu.VMEM((NCHUNK, CHUNK), jnp.int32),
        pltpu.VMEM((NCHUNK, CHUNK), jnp.int32),
        pltpu.VMEM((NCHUNK, CHUNK, HID), jnp.float32),
        pltpu.SemaphoreType.DMA,
        pltpu.SemaphoreType.DMA,
    ],
)(_sc_agg_body)


def kernel(x, edge_index, W1, b1, W2, b2):
    edges = edge_index.astype(jnp.int32).reshape(2, NW, NCHUNK, CHUNK)

    zeros = jnp.zeros((NPAD, HID), jnp.float32)
    w2p = jnp.pad(W2, ((0, 0), (0, HID - W2.shape[1])))
    w2bd = jnp.kron(jnp.eye(16, dtype=jnp.float32), w2p)
    b1r = b1.reshape(1, HID)
    b2r = jnp.tile(jnp.pad(b2, (0, HID - b2.shape[0])), 16).reshape(1, 128)

    support1 = _tc_linear(x.T, W1, b1r)
    p1 = _sc_agg(support1, edges, zeros)

    p1v = p1.reshape(NC, NROW128, 128)
    support2 = _tc_combine_linear(p1v, w2bd, b2r).reshape(NPAD, HID)
    p2 = _sc_agg(support2, edges, zeros)

    out = _tc_combine(p2.reshape(NC, NROW128, 128)).reshape(NPAD * HID)
    cols = [lax.slice(out, (k,), (k + HID * N_NODES - HID + 1,), (HID,))
            for k in range(W2.shape[1])]
    return jnp.stack(cols, axis=1)
